# baseline (speedup 1.0000x reference)
import functools

import jax
import jax.numpy as jnp
from jax import lax
from jax.experimental import pallas as pl
from jax.experimental.pallas import tpu as pltpu
from jax.experimental.pallas import tpu_sc as plsc

_B = 2048
_CODE_DIM = 32
_K = 512
_TEMP = 100000000.0
_ROWS = _B * _CODE_DIM
_BLK = 2048

_NC = 2
_NS = 16
_L = 16
_TILES = _NC * _NS
_RPT = _ROWS // _TILES
_WROWS = 64
_NWIN = _RPT // _WROWS


def _idx_body(x_ref, cb_ref, idx_ref):
    x = x_ref[:, :]
    cb = cb_ref[0, :]
    d = jnp.exp(-jnp.abs(x - cb[None, :]))
    dmax = jnp.max(d, axis=1, keepdims=True)
    iota = lax.broadcasted_iota(jnp.int32, (_BLK, _K), 1)
    idx = jnp.min(jnp.where(d == dmax, iota, _K), axis=1, keepdims=True)
    idx_ref[:, :] = idx.reshape(_BLK // 128, 128)


def _soft_body(x_ref, cb_ref, soft_ref, q_ref):
    x = x_ref[:, :]
    cb = cb_ref[0, :]
    d = jnp.exp(-jnp.abs(x - cb[None, :]))
    dmax = jnp.max(d, axis=1, keepdims=True)
    m = _TEMP * dmax
    t = _TEMP * d
    e = jnp.exp(t - m)
    s = jnp.sum(e, axis=1, keepdims=True)
    soft = e / s
    soft_ref[:, :] = soft
    q = lax.dot_general(
        soft, cb[:, None], (((1,), (0,)), ((), ())),
        preferred_element_type=jnp.float32)
    q_ref[:, :] = q.reshape(_BLK // 128, 128)


_sc_mesh = plsc.VectorSubcoreMesh(
    core_axis_name="c", subcore_axis_name="s",
    num_cores=_NC, num_subcores=_NS)


@functools.partial(
    pl.kernel,
    mesh=_sc_mesh,
    out_type=jax.ShapeDtypeStruct((_ROWS, _K), jnp.float32),
    scratch_types=[
        pltpu.VMEM((_RPT,), jnp.int32),
        pltpu.VMEM((_WROWS, _K), jnp.float32),
    ],
    compiler_params=pltpu.CompilerParams(
        use_tc_tiling_on_sc=True, needs_layout_passes=False),
    cost_estimate=pl.CostEstimate(
        flops=2 * _ROWS, transcendentals=0,
        bytes_accessed=_ROWS * _K * 4 + _ROWS * 4),
)
def _hard_sc(idx_hbm, out_hbm, idx_v, buf):
    wid = lax.axis_index("s") * _NC + lax.axis_index("c")
    base = wid * _RPT
    pltpu.sync_copy(idx_hbm.at[pl.ds(base, _RPT)], idx_v)
    zeros = jnp.zeros((_L,), jnp.float32)
    ones = jnp.full((_L,), 1.0, jnp.float32)
    iot = lax.iota(jnp.int32, _L)

    @pl.loop(0, _WROWS)
    def _zrow(r):
        @pl.loop(0, _K // _L)
        def _zcol(c):
            buf[r, pl.ds(c * _L, _L)] = zeros

    @pl.loop(0, _NWIN)
    def _win(j):
        for g in range(_WROWS // _L):
            iv = idx_v[pl.ds(j * _WROWS + g * _L, _L)]
            plsc.store_scatter(buf, [g * _L + iot, iv], ones)
        pltpu.sync_copy(buf, out_hbm.at[pl.ds(base + j * _WROWS, _WROWS), :])
        for g in range(_WROWS // _L):
            iv = idx_v[pl.ds(j * _WROWS + g * _L, _L)]
            plsc.store_scatter(buf, [g * _L + iot, iv], zeros)


@functools.partial(jax.jit)
def kernel(inputs, codebook):
    x = inputs.reshape(_ROWS, 1)
    grid = (_ROWS // _BLK,)
    idx = pl.pallas_call(
        _idx_body,
        grid=grid,
        in_specs=[
            pl.BlockSpec((_BLK, 1), lambda i: (i, 0)),
            pl.BlockSpec((1, _K), lambda i: (0, 0)),
        ],
        out_specs=pl.BlockSpec((_BLK // 128, 128), lambda i: (i, 0)),
        out_shape=jax.ShapeDtypeStruct((_ROWS // 128, 128), jnp.int32),
    )(x, codebook)
    hard = _hard_sc(idx.reshape(_ROWS))
    soft, q = pl.pallas_call(
        _soft_body,
        grid=grid,
        in_specs=[
            pl.BlockSpec((_BLK, 1), lambda i: (i, 0)),
            pl.BlockSpec((1, _K), lambda i: (0, 0)),
        ],
        out_specs=[
            pl.BlockSpec((_BLK, _K), lambda i: (i, 0)),
            pl.BlockSpec((_BLK // 128, 128), lambda i: (i, 0)),
        ],
        out_shape=[
            jax.ShapeDtypeStruct((_ROWS, _K), jnp.float32),
            jax.ShapeDtypeStruct((_ROWS // 128, 128), jnp.float32),
        ],
    )(x, codebook)
    soft = soft.reshape(_B, _CODE_DIM, _K)
    hard = hard.reshape(_B, _CODE_DIM, _K)
    q = q.reshape(_B, _CODE_DIM)
    return (soft, hard, q)


# speedup vs baseline: 1.2491x; 1.2491x over previous
import functools

import jax
import jax.numpy as jnp
from jax import lax
from jax.experimental import pallas as pl
from jax.experimental.pallas import tpu as pltpu
from jax.experimental.pallas import tpu_sc as plsc

_B = 2048
_CODE_DIM = 32
_K = 512
_TEMP = 100000000.0
_ROWS = _B * _CODE_DIM
_BLK = 2048
_R3 = _BLK // 128

_NC = 2
_NS = 16
_L = 16
_TILES = _NC * _NS
_RPT = _ROWS // _TILES
_WROWS = 64
_NWIN = _RPT // _WROWS


def _idx_body(x_ref, cb_ref, idx_ref):
    x = x_ref[:, :][:, :, None]
    cb = cb_ref[0, :][None, None, :]
    d = jnp.exp(-jnp.abs(x - cb))
    dmax = jnp.max(d, axis=2, keepdims=True)
    iota = lax.broadcasted_iota(jnp.int32, (_R3, 128, _K), 2)
    idx_ref[:, :] = jnp.min(jnp.where(d == dmax, iota, _K), axis=2)


def _soft_body(x_ref, cb_ref, soft_ref, q_ref):
    x = x_ref[:, :][:, :, None]
    cb = cb_ref[0, :][None, None, :]
    d = jnp.exp(-jnp.abs(x - cb))
    dmax = jnp.max(d, axis=2, keepdims=True)
    m = _TEMP * dmax
    t = _TEMP * d
    e = jnp.exp(t - m)
    s = jnp.sum(e, axis=2, keepdims=True)
    soft = e / s
    soft_ref[:, :] = soft.reshape(_BLK, _K)
    q_ref[:, :] = jnp.sum(soft * cb, axis=2)


_sc_mesh = plsc.VectorSubcoreMesh(
    core_axis_name="c", subcore_axis_name="s",
    num_cores=_NC, num_subcores=_NS)


@functools.partial(
    pl.kernel,
    mesh=_sc_mesh,
    out_type=jax.ShapeDtypeStruct((_ROWS, _K), jnp.float32),
    scratch_types=[
        pltpu.VMEM((_RPT,), jnp.int32),
        pltpu.VMEM((_WROWS, _K), jnp.float32),
    ],
    compiler_params=pltpu.CompilerParams(
        use_tc_tiling_on_sc=True, needs_layout_passes=False),
    cost_estimate=pl.CostEstimate(
        flops=2 * _ROWS, transcendentals=0,
        bytes_accessed=_ROWS * _K * 4 + _ROWS * 4),
)
def _hard_sc(idx_hbm, out_hbm, idx_v, buf):
    wid = lax.axis_index("s") * _NC + lax.axis_index("c")
    base = wid * _RPT
    pltpu.sync_copy(idx_hbm.at[pl.ds(base, _RPT)], idx_v)
    zeros = jnp.zeros((_L,), jnp.float32)
    ones = jnp.full((_L,), 1.0, jnp.float32)
    iot = lax.iota(jnp.int32, _L)

    @pl.loop(0, _WROWS)
    def _zrow(r):
        @pl.loop(0, _K // _L)
        def _zcol(c):
            buf[r, pl.ds(c * _L, _L)] = zeros

    @pl.loop(0, _NWIN)
    def _win(j):
        for g in range(_WROWS // _L):
            iv = idx_v[pl.ds(j * _WROWS + g * _L, _L)]
            plsc.store_scatter(buf, [g * _L + iot, iv], ones)
        pltpu.sync_copy(buf, out_hbm.at[pl.ds(base + j * _WROWS, _WROWS), :])
        for g in range(_WROWS // _L):
            iv = idx_v[pl.ds(j * _WROWS + g * _L, _L)]
            plsc.store_scatter(buf, [g * _L + iot, iv], zeros)


@functools.partial(jax.jit)
def kernel(inputs, codebook):
    x = inputs.reshape(_ROWS // 128, 128)
    grid = (_ROWS // _BLK,)
    idx = pl.pallas_call(
        _idx_body,
        grid=grid,
        in_specs=[
            pl.BlockSpec((_R3, 128), lambda i: (i, 0)),
            pl.BlockSpec((1, _K), lambda i: (0, 0)),
        ],
        out_specs=pl.BlockSpec((_R3, 128), lambda i: (i, 0)),
        out_shape=jax.ShapeDtypeStruct((_ROWS // 128, 128), jnp.int32),
    )(x, codebook)
    hard = _hard_sc(idx.reshape(_ROWS))
    soft, q = pl.pallas_call(
        _soft_body,
        grid=grid,
        in_specs=[
            pl.BlockSpec((_R3, 128), lambda i: (i, 0)),
            pl.BlockSpec((1, _K), lambda i: (0, 0)),
        ],
        out_specs=[
            pl.BlockSpec((_BLK, _K), lambda i: (i, 0)),
            pl.BlockSpec((_R3, 128), lambda i: (i, 0)),
        ],
        out_shape=[
            jax.ShapeDtypeStruct((_ROWS, _K), jnp.float32),
            jax.ShapeDtypeStruct((_ROWS // 128, 128), jnp.float32),
        ],
    )(x, codebook)
    soft = soft.reshape(_B, _CODE_DIM, _K)
    hard = hard.reshape(_B, _CODE_DIM, _K)
    q = q.reshape(_B, _CODE_DIM)
    return (soft, hard, q)


# speedup vs baseline: 1.2762x; 1.0217x over previous
import functools

import jax
import jax.numpy as jnp
from jax import lax
from jax.experimental import pallas as pl
from jax.experimental.pallas import tpu as pltpu
from jax.experimental.pallas import tpu_sc as plsc

_B = 2048
_CODE_DIM = 32
_K = 512
_TEMP = 100000000.0
_ROWS = _B * _CODE_DIM
_BLK_A = 4096
_R3A = _BLK_A // 128
_BLK = 4096
_R3 = _BLK // 128

_NC = 2
_NS = 16
_L = 16
_TILES = _NC * _NS
_RPT = _ROWS // _TILES
_WROWS = 64
_NWIN = _RPT // _WROWS


def _idx_body(x_ref, cb_ref, idx_ref):
    x = x_ref[:, :][:, :, None]
    cb = cb_ref[0, :][None, None, :]
    d = jnp.exp(-jnp.abs(x - cb))
    dmax = jnp.max(d, axis=2, keepdims=True)
    iota = lax.broadcasted_iota(jnp.int32, (_R3A, 128, _K), 2)
    idx_ref[:, :] = jnp.min(jnp.where(d == dmax, iota, _K), axis=2)


def _soft_body(x_ref, cb_ref, soft_ref, q_ref):
    x = x_ref[:, :][:, :, None]
    cb = cb_ref[0, :][None, None, :]
    d = jnp.exp(-jnp.abs(x - cb))
    dmax = jnp.max(d, axis=2, keepdims=True)
    m = _TEMP * dmax
    t = _TEMP * d
    e = jnp.exp(t - m)
    s = jnp.sum(e, axis=2, keepdims=True)
    soft = e / s
    soft_ref[:, :] = soft.reshape(_BLK, _K)
    q_ref[:, :] = jnp.sum(soft * cb, axis=2)


_sc_mesh = plsc.VectorSubcoreMesh(
    core_axis_name="c", subcore_axis_name="s",
    num_cores=_NC, num_subcores=_NS)


@functools.partial(
    pl.kernel,
    mesh=_sc_mesh,
    out_type=jax.ShapeDtypeStruct((_ROWS, _K), jnp.float32),
    scratch_types=[
        pltpu.VMEM((_RPT,), jnp.int32),
        pltpu.VMEM((_WROWS, _K), jnp.float32),
    ],
    compiler_params=pltpu.CompilerParams(
        use_tc_tiling_on_sc=True, needs_layout_passes=False),
    cost_estimate=pl.CostEstimate(
        flops=2 * _ROWS, transcendentals=0,
        bytes_accessed=_ROWS * _K * 4 + _ROWS * 4),
)
def _hard_sc(idx_hbm, out_hbm, idx_v, buf):
    wid = lax.axis_index("s") * _NC + lax.axis_index("c")
    base = wid * _RPT
    pltpu.sync_copy(idx_hbm.at[pl.ds(base, _RPT)], idx_v)
    zeros = jnp.zeros((_L,), jnp.float32)
    ones = jnp.full((_L,), 1.0, jnp.float32)
    iot = lax.iota(jnp.int32, _L)

    @pl.loop(0, _WROWS)
    def _zrow(r):
        @pl.loop(0, _K // _L)
        def _zcol(c):
            buf[r, pl.ds(c * _L, _L)] = zeros

    @pl.loop(0, _NWIN)
    def _win(j):
        for g in range(_WROWS // _L):
            iv = idx_v[pl.ds(j * _WROWS + g * _L, _L)]
            plsc.store_scatter(buf, [g * _L + iot, iv], ones)
        pltpu.sync_copy(buf, out_hbm.at[pl.ds(base + j * _WROWS, _WROWS), :])
        for g in range(_WROWS // _L):
            iv = idx_v[pl.ds(j * _WROWS + g * _L, _L)]
            plsc.store_scatter(buf, [g * _L + iot, iv], zeros)


@functools.partial(jax.jit)
def kernel(inputs, codebook):
    x = inputs.reshape(_ROWS // 128, 128)
    grid = (_ROWS // _BLK,)
    idx = pl.pallas_call(
        _idx_body,
        grid=(_ROWS // _BLK_A,),
        in_specs=[
            pl.BlockSpec((_R3A, 128), lambda i: (i, 0)),
            pl.BlockSpec((1, _K), lambda i: (0, 0)),
        ],
        out_specs=pl.BlockSpec((_R3A, 128), lambda i: (i, 0)),
        out_shape=jax.ShapeDtypeStruct((_ROWS // 128, 128), jnp.int32),
    )(x, codebook)
    hard = _hard_sc(idx.reshape(_ROWS))
    soft, q = pl.pallas_call(
        _soft_body,
        grid=grid,
        in_specs=[
            pl.BlockSpec((_R3, 128), lambda i: (i, 0)),
            pl.BlockSpec((1, _K), lambda i: (0, 0)),
        ],
        out_specs=[
            pl.BlockSpec((_BLK, _K), lambda i: (i, 0)),
            pl.BlockSpec((_R3, 128), lambda i: (i, 0)),
        ],
        out_shape=[
            jax.ShapeDtypeStruct((_ROWS, _K), jnp.float32),
            jax.ShapeDtypeStruct((_ROWS // 128, 128), jnp.float32),
        ],
    )(x, codebook)
    soft = soft.reshape(_B, _CODE_DIM, _K)
    hard = hard.reshape(_B, _CODE_DIM, _K)
    q = q.reshape(_B, _CODE_DIM)
    return (soft, hard, q)
